# baseline (speedup 1.0000x reference)
import functools

import jax
import jax.numpy as jnp
from jax import lax
from jax.experimental import pallas as pl
from jax.experimental.pallas import tpu as pltpu
from jax.experimental.pallas import tpu_sc as plsc

L = 16
PK = 4


def _mf_kernel(B, D, num_cores, num_subcores):
    NW = num_cores * num_subcores
    bpw = B // NW
    cpw = bpw // 2
    W = PK * D
    mesh = plsc.VectorSubcoreMesh(
        core_axis_name="c", subcore_axis_name="s",
        num_cores=num_cores, num_subcores=num_subcores)

    @functools.partial(
        pl.kernel,
        out_type=jax.ShapeDtypeStruct((B,), jnp.float32),
        mesh=mesh,
        scratch_types=[
            pltpu.VMEM((bpw,), jnp.int32),
            pltpu.VMEM((bpw,), jnp.int32),
            pltpu.VMEM((cpw,), jnp.int32),
            pltpu.VMEM((cpw,), jnp.int32),
            pltpu.VMEM((cpw, W), jnp.float32),
            pltpu.VMEM((cpw, W), jnp.float32),
            pltpu.VMEM((bpw,), jnp.float32),
            pltpu.VMEM((bpw,), jnp.float32),
            pltpu.SemaphoreType.DMA,
        ],
        compiler_params=pltpu.CompilerParams(
            needs_layout_passes=False, use_tc_tiling_on_sc=True),
    )
    def mf(uid_hbm, iid_hbm, ut_hbm, it_hbm, bias_hbm, out_hbm,
           uidx_v, iidx_v, upr_v, ipr_v, urows_v, irows_v, bias_v, out_v,
           sem):
        wid = lax.axis_index("s") * num_cores + lax.axis_index("c")
        base = wid * bpw

        pltpu.sync_copy(uid_hbm.at[pl.ds(base, bpw)], uidx_v)
        pltpu.sync_copy(iid_hbm.at[pl.ds(base, bpw)], iidx_v)
        cb = pltpu.async_copy(bias_hbm.at[iidx_v], bias_v, sem)

        lane = lax.iota(jnp.int32, L)

        for c in range(2):
            coff = c * cpw
            def shift(k, carry):
                upr_v[pl.ds(k * L, L)] = (
                    uidx_v[pl.ds(coff + k * L, L)] >> 2)
                ipr_v[pl.ds(k * L, L)] = (
                    iidx_v[pl.ds(coff + k * L, L)] >> 2)
                return carry
            lax.fori_loop(0, cpw // L, shift, 0)

            cu = pltpu.async_copy(ut_hbm.at[upr_v], urows_v, sem)
            ci = pltpu.async_copy(it_hbm.at[ipr_v], irows_v, sem)
            cu.wait()
            ci.wait()

            def group(g, carry):
                rvec = g * L + lane
                uq = (uidx_v[pl.ds(coff + g * L, L)] & (PK - 1)) * D
                iq = (iidx_v[pl.ds(coff + g * L, L)] & (PK - 1)) * D
                accs = [jnp.zeros((L,), jnp.float32) for _ in range(4)]
                for j in range(D):
                    dj = (lane + j) & (D - 1)
                    u = plsc.load_gather(urows_v, [rvec, uq + dj])
                    t = plsc.load_gather(irows_v, [rvec, iq + dj])
                    accs[j % 4] = accs[j % 4] + u * t
                tot = (accs[0] + accs[1]) + (accs[2] + accs[3])
                out_v[pl.ds(coff + g * L, L)] = tot
                return carry

            lax.fori_loop(0, cpw // L, group, 0)

        cb.wait()

        def addb(k, carry):
            sl = pl.ds(k * L, L)
            out_v[sl] = out_v[sl] + bias_v[sl]
            return carry
        lax.fori_loop(0, bpw // L, addb, 0)

        pltpu.sync_copy(out_v, out_hbm.at[pl.ds(base, bpw)])

    return mf


def kernel(user_ids, item_ids, user_table, item_table, item_bias_table):
    B = user_ids.shape[0]
    N, D = user_table.shape
    ut4 = user_table.reshape((N // PK, PK * D))
    it4 = item_table.reshape((item_table.shape[0] // PK, PK * D))
    bias_flat = item_bias_table.reshape((item_bias_table.shape[0],))
    mf = _mf_kernel(B, D, 2, 16)
    return mf(user_ids.astype(jnp.int32), item_ids.astype(jnp.int32),
              ut4, it4, bias_flat)
